# baseline (speedup 1.0000x reference)
import functools

import jax
import jax.numpy as jnp
from jax import lax
from jax.experimental import pallas as pl
from jax.experimental.pallas import tpu as pltpu
from jax.experimental.pallas import tpu_sc as plsc

_NC = 2
_NS = 16
_LANES = 16


def _sc_targets_body(gt_hbm, out_hbm, gtv, buf, dma_sem):
    wid = lax.axis_index("s") * _NC + lax.axis_index("c")
    b = wid // 2
    half = wid % 2
    F = out_hbm.shape[2]
    half_f = F // 2
    fbase = half * half_f
    n_gt = gt_hbm.shape[1] // 2

    pltpu.sync_copy(gt_hbm.at[b], gtv)

    gvecs = [gtv[pl.ds(k * _LANES, _LANES)]
             for k in range((2 * n_gt) // _LANES)]
    s_sc = []
    e_sc = []
    for n in range(n_gt):
        k, i = divmod(2 * n, _LANES)
        s_sc.append(jnp.clip(gvecs[k][i], 0.0, 1.0))
        e_sc.append(jnp.clip(gvecs[k][i + 1], 0.0, 1.0))

    tot_len = jnp.float32(0.0)
    for n in range(n_gt):
        tot_len = tot_len + jnp.maximum(e_sc[n] - s_sc[n], 0.0)
    ns_f = jnp.where(tot_len > 0.0, jnp.float32(1.0), jnp.float32(0.0))

    inv_f = 1.0 / F

    def body_j(j, _):
        f0 = fbase + j * _LANES
        fvec = (lax.iota(jnp.int32, _LANES) + f0).astype(jnp.float32)
        left = fvec * inv_f
        right = left + inv_f

        neg = jnp.full((_LANES,), -1.0, jnp.float32)
        zero = jnp.zeros((_LANES,), jnp.float32)
        v1, v2, v3 = neg, neg, neg
        s1 = s2 = s3 = zero
        e1 = e2 = e3 = zero
        for n in range(n_gt):
            s_n = s_sc[n]
            e_n = e_sc[n]
            o = jnp.maximum(
                jnp.minimum(e_n, right) - jnp.maximum(s_n, left), 0.0)
            c1 = o > v1
            c2 = o > v2
            c3 = o > v3
            sv = jnp.full((_LANES,), s_n, jnp.float32)
            ev = jnp.full((_LANES,), e_n, jnp.float32)
            v3 = jnp.where(c3, jnp.where(c2, v2, o), v3)
            s3 = jnp.where(c3, jnp.where(c2, s2, sv), s3)
            e3 = jnp.where(c3, jnp.where(c2, e2, ev), e3)
            v2 = jnp.where(c2, jnp.where(c1, v1, o), v2)
            s2 = jnp.where(c2, jnp.where(c1, s1, sv), s2)
            e2 = jnp.where(c2, jnp.where(c1, e1, ev), e2)
            v1 = jnp.where(c1, o, v1)
            s1 = jnp.where(c1, sv, s1)
            e1 = jnp.where(c1, ev, e1)

        one = jnp.ones((_LANES,), jnp.float32)
        sl = pl.ds(j * _LANES, _LANES)
        for p, (vp, sp_, ep) in enumerate(
                ((v1, s1, e1), (v2, s2, e2), (v3, s3, e3))):
            zp = jnp.where(vp > 0.0, one, zero) * ns_f
            buf[p, sl] = zp
            buf[3 + p, sl] = sp_ * ns_f
            buf[6 + p, sl] = ep * ns_f
        return 0

    lax.fori_loop(0, half_f // _LANES, body_j, 0)

    pltpu.async_copy(buf, out_hbm.at[b, :, pl.ds(fbase, half_f)],
                     dma_sem).wait()


def _sc_targets(gt_boxes, B, F):
    mesh = plsc.VectorSubcoreMesh(core_axis_name="c", subcore_axis_name="s")
    n_gt = gt_boxes.shape[1]
    run = pl.kernel(
        _sc_targets_body,
        mesh=mesh,
        out_type=jax.ShapeDtypeStruct((B, 9, F), jnp.float32),
        scratch_types=[
            pltpu.VMEM((2 * n_gt,), jnp.float32),
            pltpu.VMEM((9, F // 2), jnp.float32),
            pltpu.SemaphoreType.DMA,
        ],
    )
    return run(gt_boxes.reshape(gt_boxes.shape[0], 2 * n_gt))


def _loss_block(ps_ref, pe_ref, pc_ref, tgt_ref, out_ref):
    b = pl.program_id(0)
    _, Pp, _, T, F = ps_ref.shape

    main_b = jnp.float32(0.0)
    npos_b = jnp.float32(0.0)
    for p in range(Pp):
        z = tgt_ref[0, p:p + 1, :]
        ts = tgt_ref[0, 3 + p:4 + p, :]
        te = tgt_ref[0, 6 + p:7 + p, :]
        rw = 5.0 * z
        aw = 0.5 + 0.5 * z

        ps = ps_ref[0, p, 0]
        pe = pe_ref[0, p, 0]
        pc = pc_ref[0, p, 0]
        d1 = jnp.abs(ps - ts)
        m1 = jnp.minimum(d1, 1.0)
        d2 = jnp.abs(pe - te)
        m2 = jnp.minimum(d2, 1.0)
        sl = m1 * (d1 - 0.5 * m1) + m2 * (d2 - 0.5 * m2)
        sp = jnp.maximum(pc, 0.0) + jnp.log1p(jnp.exp(-jnp.abs(pc)))
        contrib = rw * sl + aw * sp - z * pc
        main_b += jnp.sum(contrib)
        npos_b += jnp.float32(T) * jnp.sum(z)

    blk = jnp.concatenate(
        [jnp.full((1, 128), main_b, jnp.float32),
         jnp.full((1, 128), npos_b, jnp.float32),
         jnp.zeros((6, 128), jnp.float32)], axis=0)

    @pl.when(b == 0)
    def _():
        out_ref[...] = blk

    @pl.when(b != 0)
    def _():
        out_ref[...] = out_ref[...] + blk


@functools.partial(jax.jit, static_argnames=())
def kernel(raw_preds, gt_boxes):
    B, Pp, C, T, F = raw_preds.shape
    targets = _sc_targets(gt_boxes, B, F)
    out = pl.pallas_call(
        _loss_block,
        grid=(B,),
        in_specs=[
            pl.BlockSpec((1, Pp, 1, T, F), lambda b: (b, 0, 0, 0, 0)),
            pl.BlockSpec((1, Pp, 1, T, F), lambda b: (b, 0, 1, 0, 0)),
            pl.BlockSpec((1, Pp, 1, T, F), lambda b: (b, 0, 2, 0, 0)),
            pl.BlockSpec((1, 9, F), lambda b: (b, 0, 0)),
        ],
        out_specs=pl.BlockSpec((8, 128), lambda b: (0, 0)),
        out_shape=jax.ShapeDtypeStruct((8, 128), jnp.float32),
    )(raw_preds, raw_preds, raw_preds, targets)
    main = out[0, 0]
    n_pos = out[1, 0]
    return main / jnp.maximum(n_pos, 1.0)


# speedup vs baseline: 2.2393x; 2.2393x over previous
import functools

import jax
import jax.numpy as jnp
from jax.experimental import pallas as pl


def _loss_block(ps_ref, pe_ref, pc_ref, gt_ref, out_ref):
    b = pl.program_id(0)
    BB, Pp, _, T, F = ps_ref.shape
    N = gt_ref.shape[1]

    main_b = jnp.float32(0.0)
    npos_b = jnp.float32(0.0)
    for bb in range(BB):
        g = gt_ref[bb]
        s = jnp.clip(g[:, 0:1], 0.0, 1.0)
        e = jnp.clip(g[:, 1:2], 0.0, 1.0)

        lane = jax.lax.broadcasted_iota(
            jnp.int32, (1, F), 1).astype(jnp.float32)
        left = lane * (1.0 / F)
        right = left + (1.0 / F)
        ov = jnp.clip(jnp.minimum(e, right) - jnp.maximum(s, left), 0.0, None)
        not_skip = jnp.sum(ov) > 0.0
        n_col = jax.lax.broadcasted_iota(jnp.int32, (N, F), 0)
        s_b = jnp.broadcast_to(s, (N, F))
        e_b = jnp.broadcast_to(e, (N, F))

        for p in range(Pp):
            m = jnp.max(ov, axis=0, keepdims=True)
            idx = jnp.min(jnp.where(ov == m, n_col, N), axis=0, keepdims=True)
            oh = n_col == idx
            ts = jnp.sum(jnp.where(oh, s_b, 0.0), axis=0, keepdims=True)
            te = jnp.sum(jnp.where(oh, e_b, 0.0), axis=0, keepdims=True)
            pos = (m > 0.0) & not_skip
            ov = jnp.where(oh, -1.0, ov)

            z = pos.astype(jnp.float32)
            rw = 5.0 * z
            aw = 0.5 + 0.5 * z

            ps = ps_ref[bb, p, 0]
            pe = pe_ref[bb, p, 0]
            pc = pc_ref[bb, p, 0]
            d1 = jnp.abs(ps - ts)
            m1 = jnp.minimum(d1, 1.0)
            d2 = jnp.abs(pe - te)
            m2 = jnp.minimum(d2, 1.0)
            sl = m1 * (d1 - 0.5 * m1) + m2 * (d2 - 0.5 * m2)
            sp = jnp.maximum(pc, 0.0) + jnp.log1p(jnp.exp(-jnp.abs(pc)))
            contrib = rw * sl + aw * sp - z * pc
            main_b += jnp.sum(contrib)
            npos_b += jnp.float32(T) * jnp.sum(z)

    blk = jnp.concatenate(
        [jnp.full((1, 128), main_b, jnp.float32),
         jnp.full((1, 128), npos_b, jnp.float32),
         jnp.zeros((6, 128), jnp.float32)], axis=0)

    @pl.when(b == 0)
    def _():
        out_ref[...] = blk

    @pl.when(b != 0)
    def _():
        out_ref[...] = out_ref[...] + blk


@functools.partial(jax.jit, static_argnames=())
def kernel(raw_preds, gt_boxes):
    B, Pp, C, T, F = raw_preds.shape
    N = gt_boxes.shape[1]
    BB = 2
    out = pl.pallas_call(
        _loss_block,
        grid=(B // BB,),
        in_specs=[
            pl.BlockSpec((BB, Pp, 1, T, F), lambda b: (b, 0, 0, 0, 0)),
            pl.BlockSpec((BB, Pp, 1, T, F), lambda b: (b, 0, 1, 0, 0)),
            pl.BlockSpec((BB, Pp, 1, T, F), lambda b: (b, 0, 2, 0, 0)),
            pl.BlockSpec((BB, N, 2), lambda b: (b, 0, 0)),
        ],
        out_specs=pl.BlockSpec((8, 128), lambda b: (0, 0)),
        out_shape=jax.ShapeDtypeStruct((8, 128), jnp.float32),
    )(raw_preds, raw_preds, raw_preds, gt_boxes)
    main = out[0, 0]
    n_pos = out[1, 0]
    return main / jnp.maximum(n_pos, 1.0)


# speedup vs baseline: 2.2953x; 1.0250x over previous
import functools

import jax
import jax.numpy as jnp
from jax.experimental import pallas as pl


def _loss_block(ps_ref, pe_ref, pc_ref, gt_ref, out_ref):
    b = pl.program_id(0)
    BB, Pp, _, T, F = ps_ref.shape
    N = gt_ref.shape[1]

    main_b = jnp.float32(0.0)
    npos_b = jnp.float32(0.0)
    for bb in range(BB):
        g = gt_ref[bb]
        s = jnp.clip(g[:, 0:1], 0.0, 1.0)
        e = jnp.clip(g[:, 1:2], 0.0, 1.0)

        lane = jax.lax.broadcasted_iota(
            jnp.int32, (1, F), 1).astype(jnp.float32)
        left = lane * (1.0 / F)
        right = left + (1.0 / F)
        ov = jnp.clip(jnp.minimum(e, right) - jnp.maximum(s, left), 0.0, None)
        not_skip = jnp.sum(ov) > 0.0
        n_col = jax.lax.broadcasted_iota(jnp.int32, (N, F), 0)
        s_b = jnp.broadcast_to(s, (N, F))
        e_b = jnp.broadcast_to(e, (N, F))

        for p in range(Pp):
            m = jnp.max(ov, axis=0, keepdims=True)
            idx = jnp.min(jnp.where(ov == m, n_col, N), axis=0, keepdims=True)
            oh = n_col == idx
            ts = jnp.sum(jnp.where(oh, s_b, 0.0), axis=0, keepdims=True)
            te = jnp.sum(jnp.where(oh, e_b, 0.0), axis=0, keepdims=True)
            pos = (m > 0.0) & not_skip
            ov = jnp.where(oh, -1.0, ov)

            z = pos.astype(jnp.float32)
            rw = 5.0 * z
            aw = 0.5 + 0.5 * z

            ps = ps_ref[bb, p, 0]
            pe = pe_ref[bb, p, 0]
            pc = pc_ref[bb, p, 0]
            d1 = jnp.abs(ps - ts)
            m1 = jnp.minimum(d1, 1.0)
            d2 = jnp.abs(pe - te)
            m2 = jnp.minimum(d2, 1.0)
            sl = m1 * (d1 - 0.5 * m1) + m2 * (d2 - 0.5 * m2)
            sp = jnp.maximum(pc, 0.0) + jnp.log1p(jnp.exp(-jnp.abs(pc)))
            contrib = rw * sl + aw * sp - z * pc
            main_b += jnp.sum(contrib)
            npos_b += jnp.float32(T) * jnp.sum(z)

    blk = jnp.concatenate(
        [jnp.full((1, 128), main_b, jnp.float32),
         jnp.full((1, 128), npos_b, jnp.float32),
         jnp.zeros((6, 128), jnp.float32)], axis=0)

    @pl.when(b == 0)
    def _():
        out_ref[...] = blk

    @pl.when(b != 0)
    def _():
        out_ref[...] = out_ref[...] + blk


@functools.partial(jax.jit, static_argnames=())
def kernel(raw_preds, gt_boxes):
    B, Pp, C, T, F = raw_preds.shape
    N = gt_boxes.shape[1]
    BB = 4
    out = pl.pallas_call(
        _loss_block,
        grid=(B // BB,),
        in_specs=[
            pl.BlockSpec((BB, Pp, 1, T, F), lambda b: (b, 0, 0, 0, 0)),
            pl.BlockSpec((BB, Pp, 1, T, F), lambda b: (b, 0, 1, 0, 0)),
            pl.BlockSpec((BB, Pp, 1, T, F), lambda b: (b, 0, 2, 0, 0)),
            pl.BlockSpec((BB, N, 2), lambda b: (b, 0, 0)),
        ],
        out_specs=pl.BlockSpec((8, 128), lambda b: (0, 0)),
        out_shape=jax.ShapeDtypeStruct((8, 128), jnp.float32),
    )(raw_preds, raw_preds, raw_preds, gt_boxes)
    main = out[0, 0]
    n_pos = out[1, 0]
    return main / jnp.maximum(n_pos, 1.0)
